# baseline (speedup 1.0000x reference)
import numpy as np
import jax
import jax.numpy as jnp
from jax import lax
from jax.experimental import pallas as pl
from jax.experimental.pallas import tpu as pltpu
from jax.experimental.pallas import tpu_sc as plsc

D = 64
SEQ = 50
BATCH = 4096
ROWS = BATCH * SEQ
NC, NS = 2, 16
NW = NC * NS
BPW = BATCH // NW
RPW = ROWS // NW
NBUF = 5
SCALE = 8.0


def _pos_const():
    depth = D / 2
    positions = np.arange(SEQ)[:, None]
    depths = np.arange(depth)[None, :] / depth
    angle_rates = 1 / 10000 ** depths
    angle_rads = positions * angle_rates
    pos = np.concatenate([np.sin(angle_rads), np.cos(angle_rads)], axis=-1)
    return jnp.asarray(pos, dtype=jnp.float32)


def _body(x_hbm, table_hbm, pos_hbm, y_hbm, idx_bl, idx_lb, pos_v,
          rows, ybufs, *sems):
    gsems = sems[:NBUF]
    ssems = sems[NBUF:]
    wid = lax.axis_index("s") * NC + lax.axis_index("c")
    base = wid * RPW
    b0col = wid * BPW
    pltpu.sync_copy(pos_hbm, pos_v)
    pltpu.sync_copy(x_hbm.at[pl.ds(base, RPW)], idx_bl)

    iota = lax.iota(jnp.int32, 16)
    iota_seq = iota * SEQ
    dvecs = [iota + (j * 16) for j in range(D // 16)]

    def reorder(l, c):
        for k in range(BPW // 16):
            v = plsc.load_gather(idx_bl, [iota_seq + (k * 16 * SEQ + l)])
            idx_lb[pl.ds(l * BPW + k * 16, 16)] = v
        return c

    lax.fori_loop(0, SEQ, reorder, 0)

    def fire_gather(c, b):
        pltpu.async_copy(table_hbm.at[idx_lb.at[pl.ds(c * BPW, BPW)]],
                         rows.at[b], gsems[b])

    def wait_gather(c, b):
        pltpu.make_async_copy(table_hbm.at[idx_lb.at[pl.ds(c * BPW, BPW)]],
                              rows.at[b], gsems[b]).wait()

    def fire_store(c, b):
        pltpu.async_copy(ybufs.at[b],
                         y_hbm.at[c].at[:, pl.ds(b0col, BPW)], ssems[b])

    def wait_store(c, b):
        pltpu.make_async_copy(ybufs.at[b],
                              y_hbm.at[c].at[:, pl.ds(b0col, BPW)],
                              ssems[b]).wait()

    def compute(c, b):
        pvs = [pos_v[c, pl.ds(j * 16, 16)] for j in range(D // 16)]

        @plsc.parallel_loop(0, BPW, unroll=8)
        def _(r):
            cb = lax.broadcast(r, (16,))
            for j in range(D // 16):
                yv = rows[b, r, pl.ds(j * 16, 16)] * SCALE + pvs[j]
                plsc.store_scatter(ybufs.at[b], [dvecs[j], cb], yv)

    for b in range(NBUF - 1):
        fire_gather(b, b)
    fire_gather(NBUF - 1, NBUF - 1)
    wait_gather(0, 0)
    compute(0, 0)
    fire_store(0, 0)

    def block(m, carry):
        for b in range(NBUF):
            c = m * NBUF + b + 1
            bb = (b + 1) % NBUF
            bp = b % NBUF
            wait_store(c - 1, bp)
            fire_gather(c + NBUF - 1, bp)
            wait_gather(c, bb)
            compute(c, bb)
            fire_store(c, bb)
        return carry

    lax.fori_loop(0, (SEQ - NBUF) // NBUF, block, 0)

    for c in range(SEQ - NBUF + 1, SEQ):
        bb = c % NBUF
        wait_store(c - 1, (c - 1) % NBUF)
        wait_gather(c, bb)
        compute(c, bb)
        fire_store(c, bb)
    wait_store(SEQ - 1, (SEQ - 1) % NBUF)


def kernel(x, table):
    mesh = plsc.VectorSubcoreMesh(core_axis_name="c", subcore_axis_name="s")
    f = pl.kernel(
        _body,
        out_type=jax.ShapeDtypeStruct((SEQ, D, BATCH), jnp.float32),
        mesh=mesh,
        scratch_types=[
            pltpu.VMEM((RPW,), jnp.int32),
            pltpu.VMEM((RPW,), jnp.int32),
            pltpu.VMEM((SEQ, D), jnp.float32),
            pltpu.VMEM((NBUF, BPW, D), jnp.float32),
            pltpu.VMEM((NBUF, D, BPW), jnp.float32),
        ] + [pltpu.SemaphoreType.DMA] * (2 * NBUF),
        compiler_params=pltpu.CompilerParams(use_tc_tiling_on_sc=False,
                                             needs_layout_passes=False),
    )
    y = f(x.reshape(-1).astype(jnp.int32), table, _pos_const())
    return jnp.transpose(y, (2, 0, 1))


# speedup vs baseline: 1.0409x; 1.0409x over previous
import numpy as np
import jax
import jax.numpy as jnp
from jax import lax
from jax.experimental import pallas as pl
from jax.experimental.pallas import tpu as pltpu
from jax.experimental.pallas import tpu_sc as plsc

D = 64
SEQ = 50
BATCH = 4096
ROWS = BATCH * SEQ
NC, NS = 2, 16
NW = NC * NS
BPW = BATCH // NW
RPW = ROWS // NW
NBUF = 5
SCALE = 8.0


def _pos_const():
    depth = D / 2
    positions = np.arange(SEQ)[:, None]
    depths = np.arange(depth)[None, :] / depth
    angle_rates = 1 / 10000 ** depths
    angle_rads = positions * angle_rates
    pos = np.concatenate([np.sin(angle_rads), np.cos(angle_rads)], axis=-1)
    return jnp.asarray(pos, dtype=jnp.float32)


def _body(x_hbm, table_hbm, pos_hbm, y_hbm, idx_bl, idx_lb, pos_v,
          rows, ybufs, *sems):
    gsems = sems[:NBUF]
    ssems = sems[NBUF:]
    wid = lax.axis_index("s") * NC + lax.axis_index("c")
    base = wid * RPW
    b0col = wid * BPW
    pltpu.sync_copy(pos_hbm, pos_v)
    pltpu.sync_copy(x_hbm.at[pl.ds(base, RPW)], idx_bl)

    iota = lax.iota(jnp.int32, 16)
    iota_seq = iota * SEQ
    dvecs = [iota + (j * 16) for j in range(D // 16)]

    def reorder(l, c):
        for k in range(BPW // 16):
            v = plsc.load_gather(idx_bl, [iota_seq + (k * 16 * SEQ + l)])
            idx_lb[pl.ds(l * BPW + k * 16, 16)] = v
        return c

    lax.fori_loop(0, SEQ, reorder, 0)

    def fire_gather(c, b):
        pltpu.async_copy(table_hbm.at[idx_lb.at[pl.ds(c * BPW, BPW)]],
                         rows.at[b], gsems[b])

    def wait_gather(c, b):
        pltpu.make_async_copy(table_hbm.at[idx_lb.at[pl.ds(c * BPW, BPW)]],
                              rows.at[b], gsems[b]).wait()

    def fire_store(c, b):
        pltpu.async_copy(ybufs.at[b], y_hbm.at[c].at[wid], ssems[b])

    def wait_store(c, b):
        pltpu.make_async_copy(ybufs.at[b], y_hbm.at[c].at[wid],
                              ssems[b]).wait()

    def compute(c, b):
        pvs = [pos_v[c, pl.ds(j * 16, 16)] for j in range(D // 16)]

        @plsc.parallel_loop(0, BPW, unroll=8)
        def _(r):
            cb = lax.broadcast(r, (16,))
            for j in range(D // 16):
                yv = rows[b, r, pl.ds(j * 16, 16)] * SCALE + pvs[j]
                plsc.store_scatter(ybufs.at[b], [dvecs[j], cb], yv)

    for b in range(NBUF - 1):
        fire_gather(b, b)
    fire_gather(NBUF - 1, NBUF - 1)
    wait_gather(0, 0)
    compute(0, 0)
    fire_store(0, 0)

    def block(m, carry):
        for b in range(NBUF):
            c = m * NBUF + b + 1
            bb = (b + 1) % NBUF
            bp = b % NBUF
            wait_store(c - 1, bp)
            fire_gather(c + NBUF - 1, bp)
            wait_gather(c, bb)
            compute(c, bb)
            fire_store(c, bb)
        return carry

    lax.fori_loop(0, (SEQ - NBUF) // NBUF, block, 0)

    for c in range(SEQ - NBUF + 1, SEQ):
        bb = c % NBUF
        wait_store(c - 1, (c - 1) % NBUF)
        wait_gather(c, bb)
        compute(c, bb)
        fire_store(c, bb)
    wait_store(SEQ - 1, (SEQ - 1) % NBUF)


def kernel(x, table):
    mesh = plsc.VectorSubcoreMesh(core_axis_name="c", subcore_axis_name="s")
    f = pl.kernel(
        _body,
        out_type=jax.ShapeDtypeStruct((SEQ, NW, D, BPW), jnp.float32),
        mesh=mesh,
        scratch_types=[
            pltpu.VMEM((RPW,), jnp.int32),
            pltpu.VMEM((RPW,), jnp.int32),
            pltpu.VMEM((SEQ, D), jnp.float32),
            pltpu.VMEM((NBUF, BPW, D), jnp.float32),
            pltpu.VMEM((NBUF, D, BPW), jnp.float32),
        ] + [pltpu.SemaphoreType.DMA] * (2 * NBUF),
        compiler_params=pltpu.CompilerParams(use_tc_tiling_on_sc=False,
                                             needs_layout_passes=False),
    )
    y = f(x.reshape(-1).astype(jnp.int32), table, _pos_const())
    return jnp.transpose(y, (1, 3, 0, 2)).reshape(BATCH, SEQ, D)
